# baseline (speedup 1.0000x reference)
import functools

import jax
import jax.numpy as jnp
from jax import lax
from jax.experimental import pallas as pl
from jax.experimental.pallas import tpu as pltpu
from jax.experimental.pallas import tpu_sc as plsc

B = 8
N = 32768
K = 2048

_SC_CORES = 2
_SC_SUBCORES = 16

_I32_MIN = jnp.iinfo(jnp.int32).min
_I32_MAX = jnp.iinfo(jnp.int32).max
_M = 0x7FFFFFFF


def _thresh_tc_body(scores_ref, ctrl_ref):
    s = scores_ref[:]
    b = lax.bitcast_convert_type(s, jnp.int32)
    key = jnp.where(b < 0, b ^ _M, b)

    lo0 = jnp.full((B, 1), _I32_MIN, jnp.int32)
    hi0 = jnp.full((B, 1), _I32_MAX, jnp.int32)

    def step(_, carry):
        lo, hi = carry
        t = hi - lo
        half = ((t >> 1) & _M) + (t & 1)
        mid = lo + half
        cnt = jnp.sum((key >= mid).astype(jnp.int32), axis=1, keepdims=True)
        ge = cnt >= K
        return jnp.where(ge, mid, lo), jnp.where(ge, hi, mid - 1)

    lo, _ = lax.fori_loop(0, 32, step, (lo0, hi0))
    t_row = lo
    g_row = jnp.sum((key > t_row).astype(jnp.int32), axis=1, keepdims=True)
    lane = lax.broadcasted_iota(jnp.int32, (B, 128), 1)
    ctrl_ref[:] = jnp.where(
        lane == 0,
        jnp.broadcast_to(t_row, (B, 128)),
        jnp.broadcast_to(g_row, (B, 128)),
    )


_thresh_tc = pl.pallas_call(
    _thresh_tc_body,
    out_shape=jax.ShapeDtypeStruct((B, 128), jnp.int32),
)


@functools.lru_cache(maxsize=None)
def _make_sc_main():
    mesh = plsc.VectorSubcoreMesh(core_axis_name="c", subcore_axis_name="s")
    n_gchunk = 2 * K // 128

    @functools.partial(
        pl.kernel,
        mesh=mesh,
        compiler_params=pltpu.CompilerParams(needs_layout_passes=False),
        out_type=(
            jax.ShapeDtypeStruct((B, K), jnp.float32),
            jax.ShapeDtypeStruct((B, 2 * K), jnp.float32),
        ),
        scratch_types=[
            pltpu.VMEM((N,), jnp.float32),
            pltpu.VMEM((128,), jnp.int32),
            pltpu.VMEM((K + 16,), jnp.int32),
            pltpu.VMEM((K + 16,), jnp.int32),
            pltpu.VMEM((K + 16,), jnp.int32),
            pltpu.VMEM((K + 16,), jnp.int32),
            pltpu.VMEM((256,), jnp.int32),
            pltpu.VMEM((256,), jnp.int32),
            pltpu.VMEM((n_gchunk, 128), jnp.int32),
            pltpu.VMEM((2 * K,), jnp.float32),
            pltpu.VMEM((K,), jnp.float32),
            pltpu.SemaphoreType.DMA,
        ],
    )
    def main_k(
        scores_hbm, ctrl_hbm, kflat_hbm, oscore_hbm, okpts_hbm,
        sv, cv, kb0, ib0, kb1, ib1, hist, base, gidx, gbuf, sbuf, sem,
    ):
        wid = lax.axis_index("s") * _SC_CORES + lax.axis_index("c")

        @pl.when(wid < B)
        def _():
            pltpu.sync_copy(scores_hbm.at[wid], sv)
            pltpu.sync_copy(ctrl_hbm.at[wid], cv)
            head = cv[pl.ds(0, 16)]
            t_vec = jnp.broadcast_to(head[0], (16,))
            g_cnt = head[1]
            lanes = lax.broadcasted_iota(jnp.int32, (16,), 0)

            def cbody(i, carry):
                gt_off, eq_off = carry
                v = sv[pl.ds(i * 16, 16)]
                bb = lax.bitcast_convert_type(v, jnp.int32)
                keyv = jnp.where(bb < 0, bb ^ _M, bb)
                skv = keyv ^ _M
                iv = lanes + i * 16
                gt = keyv > t_vec
                eq = keyv == t_vec
                ngt = jnp.sum(gt.astype(jnp.int32))
                plsc.store_compressed(kb0.at[pl.ds(gt_off, 16)], skv, mask=gt)
                plsc.store_compressed(ib0.at[pl.ds(gt_off, 16)], iv, mask=gt)
                c = plsc.cumsum(eq.astype(jnp.int32))
                keep = eq & ((eq_off + c) <= K)
                nk = jnp.sum(keep.astype(jnp.int32))
                plsc.store_compressed(kb0.at[pl.ds(eq_off, 16)], skv, mask=keep)
                plsc.store_compressed(ib0.at[pl.ds(eq_off, 16)], iv, mask=keep)
                return gt_off + ngt, eq_off + nk

            lax.fori_loop(0, N // 16, cbody, (jnp.int32(0), g_cnt))

            zeros16 = jnp.zeros((16,), jnp.int32)
            bufs = [(kb0, ib0, kb1, ib1), (kb1, ib1, kb0, ib0)]
            for p in range(4):
                src_k, src_i, dst_k, dst_i = bufs[p % 2]
                shift = 8 * p
                for j in range(16):
                    hist[pl.ds(j * 16, 16)] = zeros16

                def hbody(i, carry, src_k=src_k, shift=shift):
                    kv = src_k[pl.ds(i * 16, 16)]
                    d = (kv >> shift) & 255
                    a, last = plsc.scan_count(d)
                    plsc.addupdate_scatter(hist, [d], a, mask=last)
                    return carry

                lax.fori_loop(0, K // 16, hbody, jnp.int32(0))

                def pbody(j, run):
                    hv = hist[pl.ds(j * 16, 16)]
                    c = plsc.cumsum(hv)
                    base[pl.ds(j * 16, 16)] = (run + c) - hv
                    return run + jnp.sum(hv)

                lax.fori_loop(0, 16, pbody, jnp.int32(0))

                def mbody(i, carry, src_k=src_k, src_i=src_i,
                          dst_k=dst_k, dst_i=dst_i, shift=shift):
                    kv = src_k[pl.ds(i * 16, 16)]
                    ix = src_i[pl.ds(i * 16, 16)]
                    d = (kv >> shift) & 255
                    a, last = plsc.scan_count(d)
                    bs = plsc.load_gather(base, [d])
                    pos = bs + a - 1
                    plsc.store_scatter(dst_k, [pos], kv)
                    plsc.store_scatter(dst_i, [pos], ix)
                    plsc.addupdate_scatter(base, [d], a, mask=last)
                    return carry

                lax.fori_loop(0, K // 16, mbody, jnp.int32(0))


            rbase2 = wid * (2 * N)

            def ebody(i, carry):
                sk = kb0[pl.ds(i * 16, 16)]
                bb = jnp.where(sk < 0, sk, sk ^ _M)
                sbuf[pl.ds(i * 16, 16)] = lax.bitcast_convert_type(
                    bb, jnp.float32
                )
                ix = ib0[pl.ds(i * 16, 16)]
                e0 = rbase2 + ix * 2
                p0 = (lanes + i * 16) * 2
                plsc.store_scatter(gidx, [p0 >> 7, p0 & 127], e0)
                plsc.store_scatter(gidx, [(p0 + 1) >> 7, (p0 + 1) & 127], e0 + 1)
                return carry

            lax.fori_loop(0, K // 16, ebody, jnp.int32(0))
            pltpu.sync_copy(sbuf, oscore_hbm.at[wid])

            copies = [
                pltpu.async_copy(
                    kflat_hbm.at[gidx.at[j]], gbuf.at[pl.ds(j * 128, 128)], sem
                )
                for j in range(n_gchunk)
            ]
            for cpy in copies:
                cpy.wait()
            pltpu.sync_copy(gbuf, okpts_hbm.at[wid])

    return main_k


def kernel(keypoints, scores):
    ctrl = _thresh_tc(scores)
    top_scores, kpts_flat = _make_sc_main()(
        scores, ctrl, keypoints.reshape(2 * B * N)
    )
    return (kpts_flat.reshape(B, K, 2), top_scores)
